# baseline (speedup 1.0000x reference)
import functools

import jax
import jax.numpy as jnp
from jax import lax
from jax.experimental import pallas as pl
from jax.experimental.pallas import tpu as pltpu
from jax.experimental.pallas import tpu_sc as plsc

NUM_CORES = 2
NUM_SUBCORES = 16
NUM_WORKERS = NUM_CORES * NUM_SUBCORES
SPLITS = ((0, 128), (128, 72))


@functools.lru_cache(maxsize=None)
def _make_gather(batch: int, hist: int, d: int):
    rows_per_w = batch // NUM_WORKERS
    mesh = plsc.VectorSubcoreMesh(core_axis_name="c", subcore_axis_name="s")

    @functools.partial(
        pl.kernel,
        out_type=jax.ShapeDtypeStruct((batch, hist, d), jnp.float32),
        mesh=mesh,
        compiler_params=pltpu.CompilerParams(use_tc_tiling_on_sc=False),
        scratch_types=[
            pltpu.VMEM((rows_per_w, hist), jnp.int32),
            pltpu.VMEM((128, d), jnp.float32),
            pltpu.SemaphoreType.DMA,
        ],
    )
    def k(idx_hbm, table_hbm, out_hbm, idx_v, rows_v, sem):
        wid = lax.axis_index("s") * NUM_CORES + lax.axis_index("c")
        base = wid * rows_per_w
        pltpu.sync_copy(idx_hbm.at[pl.ds(base, rows_per_w)], idx_v)

        def body(r, carry):
            for off, width in SPLITS:
                pltpu.async_copy(
                    table_hbm.at[idx_v.at[r, pl.ds(off, width)]],
                    rows_v.at[pl.ds(0, width)], sem,
                ).wait()
                pltpu.sync_copy(rows_v.at[pl.ds(0, width)],
                                out_hbm.at[base + r, pl.ds(off, width)])
            return carry

        lax.fori_loop(0, rows_per_w, body, 0)

    return k


def kernel(stacks, table):
    batch, hist = stacks.shape
    out = _make_gather(batch, hist, table.shape[1])(
        stacks.astype(jnp.int32), table)
    return out
